# baseline (speedup 1.0000x reference)
import functools

import jax
import jax.numpy as jnp
from jax import lax
from jax.experimental import pallas as pl
from jax.experimental.pallas import tpu as pltpu
from jax.experimental.pallas import tpu_sc as plsc

NENTITY = 1000000
NRELATION = 1000
EDIM = 64
RDIM = 128
B = 1024
NEG = 256

NC = 1
NS = 16
NW = NC * NS

NTAIL = B * (NEG + 1)
TPW = NTAIL // NW
CH = 128
NFULL = TPW // CH
REM = TPW - NFULL * CH
NCHUNK = NFULL + 1
NBUF = 4
SPW = B // NW


def _body(ent, rel, orel, tidx, hidx, ridx, tail, head, relv, orelv,
          idx_v, hidx_v, ridx_v, hbuf, rbuf, obuf, gsem, wsem, ssem, vsem,
          *bufs):
    wid = lax.axis_index("s") * NC + lax.axis_index("c")
    tbase = wid * TPW
    sbase = wid * SPW

    pltpu.sync_copy(tidx.at[pl.ds(tbase, TPW)], idx_v)
    pltpu.sync_copy(hidx.at[pl.ds(sbase, SPW)], hidx_v)
    pltpu.sync_copy(ridx.at[pl.ds(sbase, SPW)], ridx_v)

    g_h = pltpu.async_copy(ent.at[hidx_v], hbuf, ssem)
    g_r = pltpu.async_copy(rel.at[ridx_v], rbuf, ssem)
    g_o = pltpu.async_copy(orel.at[ridx_v], obuf, ssem)

    def chunk_src(c):
        n = CH if c < NFULL else REM
        return ent.at[idx_v.at[pl.ds(c * CH, n)]], n

    gd = [None] * NCHUNK
    wd = [None] * NCHUNK
    for c in range(NCHUNK):
        b = c % NBUF
        if c >= NBUF:
            wd[c - NBUF].wait()
        src, n = chunk_src(c)
        gd[c] = pltpu.async_copy(src, bufs[b].at[pl.ds(0, n)], gsem)
        if c >= NBUF - 1:
            cc = c - (NBUF - 1)
            nn = CH if cc < NFULL else REM
            gd[cc].wait()
            wd[cc] = pltpu.async_copy(
                bufs[cc % NBUF].at[pl.ds(0, nn)],
                tail.at[pl.ds(tbase + cc * CH, nn)], wsem)
    for cc in range(max(0, NCHUNK - (NBUF - 1)), NCHUNK):
        nn = CH if cc < NFULL else REM
        gd[cc].wait()
        wd[cc] = pltpu.async_copy(
            bufs[cc % NBUF].at[pl.ds(0, nn)],
            tail.at[pl.ds(tbase + cc * CH, nn)], wsem)

    g_h.wait()
    g_r.wait()
    g_o.wait()
    w_h = pltpu.async_copy(hbuf, head.at[pl.ds(sbase, SPW)], vsem)
    w_r = pltpu.async_copy(rbuf, relv.at[pl.ds(sbase, SPW)], vsem)
    w_o = pltpu.async_copy(obuf, orelv.at[pl.ds(sbase, SPW)], vsem)

    for cc in range(max(0, NCHUNK - NBUF), NCHUNK):
        wd[cc].wait()
    w_h.wait()
    w_r.wait()
    w_o.wait()


@jax.jit
def _gather_all(ent, rel, orel, tidx, hidx, ridx):
    mesh = plsc.VectorSubcoreMesh(
        core_axis_name="c", subcore_axis_name="s",
        num_cores=NC, num_subcores=NS)
    f = pl.kernel(
        _body,
        out_type=[
            jax.ShapeDtypeStruct((NTAIL, EDIM), jnp.float32),
            jax.ShapeDtypeStruct((B, EDIM), jnp.float32),
            jax.ShapeDtypeStruct((B, RDIM), jnp.float32),
            jax.ShapeDtypeStruct((B, EDIM), jnp.float32),
        ],
        mesh=mesh,
        compiler_params=pltpu.CompilerParams(use_tc_tiling_on_sc=False),
        scratch_types=[
            pltpu.VMEM((TPW,), jnp.int32),
            pltpu.VMEM((SPW,), jnp.int32),
            pltpu.VMEM((SPW,), jnp.int32),
            pltpu.VMEM((SPW, EDIM), jnp.float32),
            pltpu.VMEM((SPW, RDIM), jnp.float32),
            pltpu.VMEM((SPW, EDIM), jnp.float32),
            pltpu.SemaphoreType.DMA,
            pltpu.SemaphoreType.DMA,
            pltpu.SemaphoreType.DMA,
            pltpu.SemaphoreType.DMA,
        ] + [pltpu.VMEM((CH, EDIM), jnp.float32) for _ in range(NBUF)],
    )
    return f(ent, rel, orel, tidx, hidx, ridx)


def kernel(positive, negative, entity_embedding, relation_embedding,
           origin_relation_embedding):
    positive = positive.astype(jnp.int32)
    negative = negative.astype(jnp.int32)
    tidx = jnp.concatenate([positive[:, 2:3], negative], axis=1).reshape(-1)
    hidx = positive[:, 0]
    ridx = positive[:, 1]
    tail, head, relv, orelv = _gather_all(
        entity_embedding, relation_embedding, origin_relation_embedding,
        tidx, hidx, ridx)
    return (head[:, None, :],
            relv[:, None, :],
            tail.reshape(B, NEG + 1, EDIM),
            orelv[:, None, :])


# speedup vs baseline: 1.0117x; 1.0117x over previous
import functools

import jax
import jax.numpy as jnp
from jax import lax
from jax.experimental import pallas as pl
from jax.experimental.pallas import tpu as pltpu
from jax.experimental.pallas import tpu_sc as plsc

NENTITY = 1000000
NRELATION = 1000
EDIM = 64
RDIM = 128
B = 1024
NEG = 256

NC = 2
NS = 16
NW = NC * NS

NTAIL = B * (NEG + 1)
TPW = NTAIL // NW
CH = 128
NFULL = TPW // CH
REM = TPW - NFULL * CH
NCHUNK = NFULL + 1
NBUF = 4
SPW = B // NW


def _body(ent, rel, orel, tidx, hidx, ridx, tail, head, relv, orelv,
          idx_v, hidx_v, ridx_v, hbuf, rbuf, obuf, gsem, wsem, ssem, vsem,
          *bufs):
    wid = lax.axis_index("s") * NC + lax.axis_index("c")
    tbase = wid * TPW
    sbase = wid * SPW

    pltpu.sync_copy(tidx.at[pl.ds(tbase, TPW)], idx_v)
    pltpu.sync_copy(hidx.at[pl.ds(sbase, SPW)], hidx_v)
    pltpu.sync_copy(ridx.at[pl.ds(sbase, SPW)], ridx_v)

    g_h = pltpu.async_copy(ent.at[hidx_v], hbuf, ssem)
    g_r = pltpu.async_copy(rel.at[ridx_v], rbuf, ssem)
    g_o = pltpu.async_copy(orel.at[ridx_v], obuf, ssem)

    def chunk_src(c):
        n = CH if c < NFULL else REM
        return ent.at[idx_v.at[pl.ds(c * CH, n)]], n

    gd = [None] * NCHUNK
    wd = [None] * NCHUNK
    for c in range(NCHUNK):
        b = c % NBUF
        if c >= NBUF:
            wd[c - NBUF].wait()
        src, n = chunk_src(c)
        gd[c] = pltpu.async_copy(src, bufs[b].at[pl.ds(0, n)], gsem)
        if c >= NBUF - 1:
            cc = c - (NBUF - 1)
            nn = CH if cc < NFULL else REM
            gd[cc].wait()
            wd[cc] = pltpu.async_copy(
                bufs[cc % NBUF].at[pl.ds(0, nn)],
                tail.at[pl.ds(tbase + cc * CH, nn)], wsem)
    for cc in range(max(0, NCHUNK - (NBUF - 1)), NCHUNK):
        nn = CH if cc < NFULL else REM
        gd[cc].wait()
        wd[cc] = pltpu.async_copy(
            bufs[cc % NBUF].at[pl.ds(0, nn)],
            tail.at[pl.ds(tbase + cc * CH, nn)], wsem)

    g_h.wait()
    g_r.wait()
    g_o.wait()
    w_h = pltpu.async_copy(hbuf, head.at[pl.ds(sbase, SPW)], vsem)
    w_r = pltpu.async_copy(rbuf, relv.at[pl.ds(sbase, SPW)], vsem)
    w_o = pltpu.async_copy(obuf, orelv.at[pl.ds(sbase, SPW)], vsem)

    for cc in range(max(0, NCHUNK - NBUF), NCHUNK):
        wd[cc].wait()
    w_h.wait()
    w_r.wait()
    w_o.wait()


@jax.jit
def _gather_all(ent, rel, orel, tidx, hidx, ridx):
    mesh = plsc.VectorSubcoreMesh(
        core_axis_name="c", subcore_axis_name="s",
        num_cores=NC, num_subcores=NS)
    f = pl.kernel(
        _body,
        out_type=[
            jax.ShapeDtypeStruct((NTAIL, EDIM), jnp.float32),
            jax.ShapeDtypeStruct((B, EDIM), jnp.float32),
            jax.ShapeDtypeStruct((B, RDIM), jnp.float32),
            jax.ShapeDtypeStruct((B, EDIM), jnp.float32),
        ],
        mesh=mesh,
        compiler_params=pltpu.CompilerParams(use_tc_tiling_on_sc=False),
        scratch_types=[
            pltpu.VMEM((TPW,), jnp.int32),
            pltpu.VMEM((SPW,), jnp.int32),
            pltpu.VMEM((SPW,), jnp.int32),
            pltpu.VMEM((SPW, EDIM), jnp.float32),
            pltpu.VMEM((SPW, RDIM), jnp.float32),
            pltpu.VMEM((SPW, EDIM), jnp.float32),
            pltpu.SemaphoreType.DMA,
            pltpu.SemaphoreType.DMA,
            pltpu.SemaphoreType.DMA,
            pltpu.SemaphoreType.DMA,
        ] + [pltpu.VMEM((CH, EDIM), jnp.float32) for _ in range(NBUF)],
    )
    return f(ent, rel, orel, tidx, hidx, ridx)


def _as128(x):
    n, d = x.shape
    y = jax.lax.optimization_barrier(x.reshape(n // 2, 2 * d))
    return y.reshape(n, d)


def kernel(positive, negative, entity_embedding, relation_embedding,
           origin_relation_embedding):
    positive = positive.astype(jnp.int32)
    negative = negative.astype(jnp.int32)
    tidx = jnp.concatenate([positive[:, 2:3], negative], axis=1).reshape(-1)
    hidx = positive[:, 0]
    ridx = positive[:, 1]
    tail, head, relv, orelv = _gather_all(
        _as128(entity_embedding), relation_embedding,
        _as128(origin_relation_embedding), tidx, hidx, ridx)
    tail = jax.lax.optimization_barrier(
        tail.reshape(NTAIL // 2, 2 * EDIM)).reshape(B, NEG + 1, EDIM)
    head = jax.lax.optimization_barrier(
        head.reshape(B // 2, 2 * EDIM)).reshape(B, EDIM)
    orelv = jax.lax.optimization_barrier(
        orelv.reshape(B // 2, 2 * EDIM)).reshape(B, EDIM)
    return (head[:, None, :],
            relv[:, None, :],
            tail,
            orelv[:, None, :])
